# Initial kernel scaffold; baseline (speedup 1.0000x reference)
import jax
import jax.numpy as jnp
from jax.experimental import pallas as pl


def kernel(h, e, edge_index, Wh, bh, We, be, Wl, bl, gamma, beta, Wm0, bm0, Wm1, bm1, Wm2, bm2):
    raise NotImplementedError("write your pallas kernel here")


# speedup vs baseline: 3.2103x; 3.2103x over previous
import functools

import jax
import jax.numpy as jnp
from jax import lax
from jax.experimental import pallas as pl
from jax.experimental.pallas import tpu as pltpu
from jax.experimental.pallas import tpu_sc as plsc

N_NODES = 10000
N_EDGES = 320000
D = 128

NC = 2
NS = 16
NW = NC * NS

CHUNK = 128
N_CHUNKS = N_EDGES // CHUNK
BASE_CH = N_CHUNKS // NW
EXTRA_CH = N_CHUNKS % NW

N_PAD = 10240
ROWS_PER_TILE = N_PAD // NS
DEG_W = 16

_SC_MESH = plsc.VectorSubcoreMesh(core_axis_name="c", subcore_axis_name="s")


@functools.partial(
    pl.kernel,
    out_type=jax.ShapeDtypeStruct((NC, N_PAD, DEG_W), jnp.float32),
    mesh=_SC_MESH,
    scratch_types=[
        pltpu.VMEM((CHUNK,), jnp.int32),
        pltpu.VMEM((CHUNK, DEG_W), jnp.float32),
        pltpu.VMEM_SHARED((N_PAD, DEG_W), jnp.float32),
    ],
)
def _sc_deg(dst_hbm, zdeg_hbm, out_hbm, dst_i, ones_v, deg_sh):
    c = lax.axis_index("c")
    s = lax.axis_index("s")
    wid = s * NC + c

    ones16 = jnp.ones((16,), jnp.float32)

    def ones_body(i, cc):
        ones_v[i, :] = ones16
        return cc

    lax.fori_loop(0, CHUNK, ones_body, 0)

    r0 = s * ROWS_PER_TILE
    pltpu.sync_copy(zdeg_hbm.at[pl.ds(r0, ROWS_PER_TILE)],
                    deg_sh.at[pl.ds(r0, ROWS_PER_TILE)])
    plsc.subcore_barrier()

    nj = BASE_CH + jnp.where(wid < EXTRA_CH, 1, 0)

    def body(j, cc):
        ch = wid + NW * j
        pltpu.sync_copy(dst_hbm.at[pl.ds(ch * CHUNK, CHUNK)], dst_i)
        pltpu.sync_copy(ones_v, deg_sh.at[dst_i], add=True)
        return cc

    lax.fori_loop(0, nj, body, 0)
    plsc.subcore_barrier()
    pltpu.sync_copy(deg_sh.at[pl.ds(r0, ROWS_PER_TILE)],
                    out_hbm.at[c, pl.ds(r0, ROWS_PER_TILE)])


@functools.partial(
    pl.kernel,
    out_type=jax.ShapeDtypeStruct((NC, N_PAD, D), jnp.float32),
    mesh=_SC_MESH,
    scratch_types=[
        pltpu.VMEM((CHUNK,), jnp.int32),
        pltpu.VMEM((CHUNK,), jnp.int32),
        pltpu.VMEM((CHUNK, D), jnp.float32),
        pltpu.VMEM((CHUNK, D), jnp.float32),
        pltpu.VMEM_SHARED((N_PAD, D), jnp.float32),
        pltpu.SemaphoreType.DMA,
    ],
)
def _sc_agg(h_hbm, sig_hbm, src_hbm, dst_hbm, zrow_hbm, out_hbm,
            src_i, dst_i, rows_v, sig_v, agg_sh, sem):
    c = lax.axis_index("c")
    s = lax.axis_index("s")
    wid = s * NC + c

    r0 = s * ROWS_PER_TILE
    pltpu.sync_copy(zrow_hbm.at[pl.ds(r0, ROWS_PER_TILE)],
                    agg_sh.at[pl.ds(r0, ROWS_PER_TILE)])
    plsc.subcore_barrier()

    nj = BASE_CH + jnp.where(wid < EXTRA_CH, 1, 0)

    def body(j, cc):
        ch = wid + NW * j
        base = ch * CHUNK
        pltpu.sync_copy(src_hbm.at[pl.ds(base, CHUNK)], src_i)
        pltpu.sync_copy(dst_hbm.at[pl.ds(base, CHUNK)], dst_i)
        pltpu.sync_copy(sig_hbm.at[pl.ds(base, CHUNK)], sig_v)
        pltpu.async_copy(h_hbm.at[src_i], rows_v, sem).wait()

        def mrow(r, rc):
            for q in range(D // 16):
                sl = pl.ds(q * 16, 16)
                rows_v[r, sl] = rows_v[r, sl] * sig_v[r, sl]
            return rc

        lax.fori_loop(0, CHUNK, mrow, 0)
        pltpu.sync_copy(rows_v, agg_sh.at[dst_i], add=True)
        return cc

    lax.fori_loop(0, nj, body, 0)
    plsc.subcore_barrier()
    pltpu.sync_copy(agg_sh.at[pl.ds(r0, ROWS_PER_TILE)],
                    out_hbm.at[c, pl.ds(r0, ROWS_PER_TILE)])


def _h0_body(h_ref, w_ref, b_ref, o_ref):
    o_ref[...] = (jnp.dot(h_ref[...], w_ref[...],
                          preferred_element_type=jnp.float32) + b_ref[...])


def _sig_body(e_ref, w_ref, b_ref, o_ref):
    z = (jnp.dot(e_ref[...], w_ref[...],
                 preferred_element_type=jnp.float32) + b_ref[...])
    o_ref[...] = 1.0 / (1.0 + jnp.exp(-z))


def _layer_core(h, agg0, agg1, invd, w1, w2, b, g, bt):
    agg = (agg0 + agg1) * invd
    z = (jnp.dot(h, w1, preferred_element_type=jnp.float32)
         + jnp.dot(agg, w2, preferred_element_type=jnp.float32) + b)
    nrm = jnp.sqrt(jnp.sum(z * z, axis=1, keepdims=True))
    z = z / (nrm + 1e-12)
    z = jnp.maximum(z, 0.0)
    mu = jnp.mean(z, axis=0, keepdims=True)
    dv = z - mu
    var = jnp.mean(dv * dv, axis=0, keepdims=True)
    hn = g * dv * lax.rsqrt(var + 1e-5) + bt
    return h + hn


def _layer_body(h_ref, agg_ref, invd_ref, w1_ref, w2_ref, b_ref, g_ref,
                bt_ref, o_ref):
    o_ref[...] = _layer_core(h_ref[...], agg_ref[0], agg_ref[1], invd_ref[...],
                             w1_ref[...], w2_ref[...], b_ref[...], g_ref[...],
                             bt_ref[...])


def _last_body(h_ref, agg_ref, invd_ref, w1_ref, w2_ref, b_ref, g_ref,
               bt_ref, wm0_ref, bm0_ref, wm1_ref, bm1_ref, wm2_ref, bm2_ref,
               o_ref):
    hN = _layer_core(h_ref[...], agg_ref[0], agg_ref[1], invd_ref[...],
                     w1_ref[...], w2_ref[...], b_ref[...], g_ref[...],
                     bt_ref[...])
    hg = jnp.mean(hN, axis=0, keepdims=True)
    x = jnp.maximum(jnp.dot(hg, wm0_ref[...],
                            preferred_element_type=jnp.float32)
                    + bm0_ref[...], 0.0)
    x = jnp.maximum(jnp.dot(x, wm1_ref[...],
                            preferred_element_type=jnp.float32)
                    + bm1_ref[...], 0.0)
    o_ref[...] = (jnp.dot(x, wm2_ref[...], preferred_element_type=jnp.float32)
                  + bm2_ref[...])


_SIG_RB = 3200


def kernel(h, e, edge_index, Wh, bh, We, be, Wl, bl, gamma, beta,
           Wm0, bm0, Wm1, bm1, Wm2, bm2):
    src = edge_index[0]
    dst = edge_index[1]

    h0 = pl.pallas_call(
        _h0_body,
        out_shape=jax.ShapeDtypeStruct((N_NODES, D), jnp.float32),
    )(h, Wh, bh.reshape(1, D))

    sig = pl.pallas_call(
        _sig_body,
        grid=(N_EDGES // _SIG_RB,),
        in_specs=[
            pl.BlockSpec((_SIG_RB, 16), lambda i: (i, 0)),
            pl.BlockSpec((16, D), lambda i: (0, 0)),
            pl.BlockSpec((1, D), lambda i: (0, 0)),
        ],
        out_specs=pl.BlockSpec((_SIG_RB, D), lambda i: (i, 0)),
        out_shape=jax.ShapeDtypeStruct((N_EDGES, D), jnp.float32),
    )(e, We, be.reshape(1, D))

    zrow = jnp.zeros((N_PAD, D), jnp.float32)
    zdeg = jnp.zeros((N_PAD, DEG_W), jnp.float32)

    degp = _sc_deg(dst, zdeg)
    deg = jnp.maximum(degp[0, :N_NODES, 0] + degp[1, :N_NODES, 0], 1.0)
    invd = (1.0 / deg)[:, None]

    hc = h0
    for i in range(4):
        agg2 = _sc_agg(hc, sig, src, dst, zrow)[:, :N_NODES, :]
        w1 = Wl[i, :D]
        w2 = Wl[i, D:]
        b2 = bl[i].reshape(1, D)
        g2 = gamma[i].reshape(1, D)
        bt2 = beta[i].reshape(1, D)
        if i < 3:
            hc = pl.pallas_call(
                _layer_body,
                out_shape=jax.ShapeDtypeStruct((N_NODES, D), jnp.float32),
            )(hc, agg2, invd, w1, w2, b2, g2, bt2)
        else:
            scores = pl.pallas_call(
                _last_body,
                out_shape=jax.ShapeDtypeStruct((1, 10), jnp.float32),
            )(hc, agg2, invd, w1, w2, b2, g2, bt2,
              Wm0, bm0.reshape(1, -1), Wm1, bm1.reshape(1, -1),
              Wm2, bm2.reshape(1, -1))
    return scores
